# baseline (speedup 1.0000x reference)
import functools

import jax
import jax.numpy as jnp
from jax import lax
from jax.experimental import pallas as pl
from jax.experimental.pallas import tpu as pltpu
from jax.experimental.pallas import tpu_sc as plsc

N_NODES = 50000
N_EDGES = 800000
D_IN = 128
D_HID = 64
HALF = 32
N_GRAPHS = 64

N_TILES = 16
N_CORES = 2
ROWS_PER_TILE = N_NODES // N_TILES
E_PER_TILE = N_EDGES // N_TILES
E_CHUNK = 400
N_CHUNKS = E_PER_TILE // E_CHUNK
N_PAIRS = (N_CHUNKS + 1) // 2

E_PER_CTILE = N_EDGES // (N_CORES * N_TILES)
EC_CHUNK = 200
NC_CHUNKS = E_PER_CTILE // EC_CHUNK

NB = 2000
N_BLOCKS = N_NODES // NB

_MESH = plsc.VectorSubcoreMesh(core_axis_name="c", subcore_axis_name="s",
                               num_cores=N_CORES, num_subcores=N_TILES)


def _sc_counts_body(dst, zrows, ones32, cnt0, cnt1, dst_v, ones_v, cnt_sh):
    cid = lax.axis_index("c")
    sid = lax.axis_index("s")
    base = sid * ROWS_PER_TILE

    pltpu.sync_copy(zrows.at[pl.ds(base, ROWS_PER_TILE)],
                    cnt_sh.at[pl.ds(base, ROWS_PER_TILE)])
    pltpu.sync_copy(ones32, ones_v)
    plsc.subcore_barrier()

    ebase0 = (cid * N_TILES + sid) * E_PER_CTILE

    def chunk(c, carry):
        pltpu.sync_copy(dst.at[pl.ds(ebase0 + c * EC_CHUNK, EC_CHUNK)], dst_v)
        pltpu.sync_copy(ones_v, cnt_sh.at[dst_v], add=True)
        return carry

    lax.fori_loop(0, NC_CHUNKS, chunk, 0)
    plsc.subcore_barrier()

    @pl.when(cid == 0)
    def _():
        pltpu.sync_copy(cnt_sh.at[pl.ds(base, ROWS_PER_TILE)],
                        cnt0.at[pl.ds(base, ROWS_PER_TILE)])

    @pl.when(cid == 1)
    def _():
        pltpu.sync_copy(cnt_sh.at[pl.ds(base, ROWS_PER_TILE)],
                        cnt1.at[pl.ds(base, ROWS_PER_TILE)])


_sc_counts = functools.partial(
    pl.kernel,
    out_type=(
        jax.ShapeDtypeStruct((N_NODES, HALF), jnp.float32),
        jax.ShapeDtypeStruct((N_NODES, HALF), jnp.float32),
    ),
    mesh=_MESH,
    scratch_types=[
        pltpu.VMEM((EC_CHUNK,), jnp.int32),
        pltpu.VMEM((EC_CHUNK, HALF), jnp.float32),
        pltpu.VMEM_SHARED((N_NODES, HALF), jnp.float32),
    ],
    compiler_params=pltpu.CompilerParams(use_tc_tiling_on_sc=False),
)(_sc_counts_body)


def _sc_agg_body(ha, hb, src, dst, zrows,
                 suma, sumb,
                 idx_v0, idx_v1, dst_v0, dst_v1, rows_v0, rows_v1,
                 acc_sh, gsem0, gsem1, ssem0, ssem1):
    cid = lax.axis_index("c")
    sid = lax.axis_index("s")
    base = sid * ROWS_PER_TILE

    pltpu.sync_copy(zrows.at[pl.ds(base, ROWS_PER_TILE)],
                    acc_sh.at[pl.ds(base, ROWS_PER_TILE)])
    plsc.subcore_barrier()

    ebase0 = sid * E_PER_TILE

    def load_idx(c, idx_v, dst_v):
        eb = ebase0 + c * E_CHUNK
        pltpu.sync_copy(src.at[pl.ds(eb, E_CHUNK)], idx_v)
        pltpu.sync_copy(dst.at[pl.ds(eb, E_CHUNK)], dst_v)

    def start_gather(idx_v, rows_v, gsem):
        @pl.when(cid == 0)
        def _():
            pltpu.async_copy(ha.at[idx_v], rows_v, gsem)

        @pl.when(cid == 1)
        def _():
            pltpu.async_copy(hb.at[idx_v], rows_v, gsem)

    def wait_gather(idx_v, rows_v, gsem):
        @pl.when(cid == 0)
        def _():
            pltpu.make_async_copy(ha.at[idx_v], rows_v, gsem).wait()

        @pl.when(cid == 1)
        def _():
            pltpu.make_async_copy(hb.at[idx_v], rows_v, gsem).wait()

    bufs = ((idx_v0, dst_v0, rows_v0, gsem0, ssem0),
            (idx_v1, dst_v1, rows_v1, gsem1, ssem1))

    load_idx(0, idx_v0, dst_v0)
    start_gather(idx_v0, rows_v0, gsem0)

    def step(c, b, bo):
        idx_v, dst_v, rows_v, gsem, ssem = b
        idx_o, dst_o, rows_o, gsem_o, ssem_o = bo

        wait_gather(idx_v, rows_v, gsem)
        pltpu.async_copy(rows_v, acc_sh.at[dst_v], ssem, add=True)

        @pl.when(c + 1 < N_CHUNKS)
        def _():
            @pl.when(c >= 1)
            def _():
                pltpu.make_async_copy(rows_o, acc_sh.at[dst_o], ssem_o).wait()
            load_idx(c + 1, idx_o, dst_o)
            start_gather(idx_o, rows_o, gsem_o)

    def pair(i, carry):
        c0 = 2 * i
        step(c0, bufs[0], bufs[1])

        @pl.when(c0 + 1 < N_CHUNKS)
        def _():
            step(c0 + 1, bufs[1], bufs[0])

        return carry

    lax.fori_loop(0, N_PAIRS, pair, 0)
    pltpu.make_async_copy(rows_v1, acc_sh.at[dst_v1], ssem1).wait()
    pltpu.make_async_copy(rows_v0, acc_sh.at[dst_v0], ssem0).wait()
    plsc.subcore_barrier()

    @pl.when(cid == 0)
    def _():
        pltpu.sync_copy(acc_sh.at[pl.ds(base, ROWS_PER_TILE)],
                        suma.at[pl.ds(base, ROWS_PER_TILE)])

    @pl.when(cid == 1)
    def _():
        pltpu.sync_copy(acc_sh.at[pl.ds(base, ROWS_PER_TILE)],
                        sumb.at[pl.ds(base, ROWS_PER_TILE)])


_sc_agg = functools.partial(
    pl.kernel,
    out_type=(
        jax.ShapeDtypeStruct((N_NODES, HALF), jnp.float32),
        jax.ShapeDtypeStruct((N_NODES, HALF), jnp.float32),
    ),
    mesh=_MESH,
    scratch_types=[
        pltpu.VMEM((E_CHUNK,), jnp.int32),
        pltpu.VMEM((E_CHUNK,), jnp.int32),
        pltpu.VMEM((E_CHUNK,), jnp.int32),
        pltpu.VMEM((E_CHUNK,), jnp.int32),
        pltpu.VMEM((E_CHUNK, HALF), jnp.float32),
        pltpu.VMEM((E_CHUNK, HALF), jnp.float32),
        pltpu.VMEM_SHARED((N_NODES, HALF), jnp.float32),
        pltpu.SemaphoreType.DMA,
        pltpu.SemaphoreType.DMA,
        pltpu.SemaphoreType.DMA,
        pltpu.SemaphoreType.DMA,
    ],
    compiler_params=pltpu.CompilerParams(use_tc_tiling_on_sc=False),
)(_sc_agg_body)


def _enc_body(x_ref, w1, b1, w2, b2, oa, ob):
    h = jnp.dot(x_ref[...], w1[...], preferred_element_type=jnp.float32)
    h = jnp.maximum(h + b1[...], 0.0)
    h = jnp.dot(h, w2[...], preferred_element_type=jnp.float32)
    h = jnp.maximum(h + b2[...], 0.0)
    oa[...] = h[:, :HALF]
    ob[...] = h[:, HALF:]


def _encoder(x, w1, b1, w2, b2):
    return pl.pallas_call(
        _enc_body,
        grid=(N_BLOCKS,),
        in_specs=[
            pl.BlockSpec((NB, D_IN), lambda i: (i, 0)),
            pl.BlockSpec((D_IN, HALF), lambda i: (0, 0)),
            pl.BlockSpec((1, HALF), lambda i: (0, 0)),
            pl.BlockSpec((HALF, D_HID), lambda i: (0, 0)),
            pl.BlockSpec((1, D_HID), lambda i: (0, 0)),
        ],
        out_specs=[
            pl.BlockSpec((NB, HALF), lambda i: (i, 0)),
            pl.BlockSpec((NB, HALF), lambda i: (i, 0)),
        ],
        out_shape=[
            jax.ShapeDtypeStruct((N_NODES, HALF), jnp.float32),
            jax.ShapeDtypeStruct((N_NODES, HALF), jnp.float32),
        ],
    )(x, w1, b1, w2, b2)


def _sage_mix(sa, sb, c0, c1, ha, hb, wl, bl, wr):
    r = 1.0 / jnp.maximum(c0 + c1, 1.0)
    h = (jnp.dot(sa * r, wl[:HALF], preferred_element_type=jnp.float32)
         + jnp.dot(sb * r, wl[HALF:], preferred_element_type=jnp.float32)
         + bl
         + jnp.dot(ha, wr[:HALF], preferred_element_type=jnp.float32)
         + jnp.dot(hb, wr[HALF:], preferred_element_type=jnp.float32))
    return h


def _layer_body(sa, sb, c0, c1, ha, hb, wl, bl, wr, oa, ob):
    h = jnp.maximum(
        _sage_mix(sa[...], sb[...], c0[...], c1[...], ha[...], hb[...],
                  wl[...], bl[...], wr[...]), 0.0)
    oa[...] = h[:, :HALF]
    ob[...] = h[:, HALF:]


def _layer(sa, sb, c0, c1, ha, hb, wl, bl2, wr):
    return pl.pallas_call(
        _layer_body,
        grid=(N_BLOCKS,),
        in_specs=[
            pl.BlockSpec((NB, HALF), lambda i: (i, 0)),
            pl.BlockSpec((NB, HALF), lambda i: (i, 0)),
            pl.BlockSpec((NB, HALF), lambda i: (i, 0)),
            pl.BlockSpec((NB, HALF), lambda i: (i, 0)),
            pl.BlockSpec((NB, HALF), lambda i: (i, 0)),
            pl.BlockSpec((NB, HALF), lambda i: (i, 0)),
            pl.BlockSpec((D_HID, D_HID), lambda i: (0, 0)),
            pl.BlockSpec((1, D_HID), lambda i: (0, 0)),
            pl.BlockSpec((D_HID, D_HID), lambda i: (0, 0)),
        ],
        out_specs=[
            pl.BlockSpec((NB, HALF), lambda i: (i, 0)),
            pl.BlockSpec((NB, HALF), lambda i: (i, 0)),
        ],
        out_shape=[
            jax.ShapeDtypeStruct((N_NODES, HALF), jnp.float32),
            jax.ShapeDtypeStruct((N_NODES, HALF), jnp.float32),
        ],
    )(sa, sb, c0, c1, ha, hb, wl, bl2, wr)


def _final_body(bounds, sa, sb, c0, c1, ha, hb, w3l, b3l, w3r,
                dw1, db1, dw2, db2, out_ref):
    i = pl.program_id(0)
    row0 = i * NB
    h = jnp.maximum(
        _sage_mix(sa[...], sb[...], c0[...], c1[...], ha[...], hb[...],
                  w3l[...], b3l[...], w3r[...]), 0.0)
    d = jnp.maximum(
        jnp.dot(h, dw1[...], preferred_element_type=jnp.float32) + db1[...],
        0.0)
    o = jnp.dot(d, dw2[...], preferred_element_type=jnp.float32) + db2[...]

    @pl.when(i == 0)
    def _():
        out_ref[...] = jnp.full((N_GRAPHS, 48), -jnp.inf, jnp.float32)

    riota = lax.broadcasted_iota(jnp.int32, (NB, 1), 0) + row0
    for g in range(N_GRAPHS):
        s = bounds[g]
        e = bounds[g + 1]

        @pl.when((s < row0 + NB) & (e > row0))
        def _():
            m = jnp.max(jnp.where((riota >= s) & (riota < e), o, -jnp.inf),
                        axis=0, keepdims=True)
            out_ref[pl.ds(g, 1), :] = jnp.maximum(out_ref[pl.ds(g, 1), :], m)


def _final(bounds, sa, sb, c0, c1, ha, hb, w3l, b3l2, w3r,
           dw1, db12, dw2, db22):
    return pl.pallas_call(
        _final_body,
        grid_spec=pltpu.PrefetchScalarGridSpec(
            num_scalar_prefetch=1,
            grid=(N_BLOCKS,),
            in_specs=[
                pl.BlockSpec((NB, HALF), lambda i, b: (i, 0)),
                pl.BlockSpec((NB, HALF), lambda i, b: (i, 0)),
                pl.BlockSpec((NB, HALF), lambda i, b: (i, 0)),
                pl.BlockSpec((NB, HALF), lambda i, b: (i, 0)),
                pl.BlockSpec((NB, HALF), lambda i, b: (i, 0)),
                pl.BlockSpec((NB, HALF), lambda i, b: (i, 0)),
                pl.BlockSpec((D_HID, D_HID), lambda i, b: (0, 0)),
                pl.BlockSpec((1, D_HID), lambda i, b: (0, 0)),
                pl.BlockSpec((D_HID, D_HID), lambda i, b: (0, 0)),
                pl.BlockSpec((D_HID, HALF), lambda i, b: (0, 0)),
                pl.BlockSpec((1, HALF), lambda i, b: (0, 0)),
                pl.BlockSpec((HALF, 48), lambda i, b: (0, 0)),
                pl.BlockSpec((1, 48), lambda i, b: (0, 0)),
            ],
            out_specs=pl.BlockSpec((N_GRAPHS, 48), lambda i, b: (0, 0)),
        ),
        out_shape=jax.ShapeDtypeStruct((N_GRAPHS, 48), jnp.float32),
    )(bounds, sa, sb, c0, c1, ha, hb, w3l, b3l2, w3r, dw1, db12, dw2, db22)


def kernel(x, edge_index, batch, enc_W1, enc_b1, enc_W2, enc_b2,
           W1l, b1l, W1r, W2l, b2l, W2r, W3l, b3l, W3r,
           dec_W1, dec_b1, dec_W2, dec_b2):
    src = edge_index[0]
    dst = edge_index[1]
    zrows = jnp.zeros((N_NODES, HALF), jnp.float32)
    ones32 = jnp.ones((EC_CHUNK, HALF), jnp.float32)
    bounds = jnp.searchsorted(
        batch, jnp.arange(N_GRAPHS + 1, dtype=jnp.int32)).astype(jnp.int32)

    cnt0, cnt1 = _sc_counts(dst, zrows, ones32)

    ha, hb = _encoder(x, enc_W1, enc_b1.reshape(1, -1),
                      enc_W2, enc_b2.reshape(1, -1))

    sa, sb = _sc_agg(ha, hb, src, dst, zrows)
    ha, hb = _layer(sa, sb, cnt0, cnt1, ha, hb, W1l, b1l.reshape(1, -1), W1r)

    sa, sb = _sc_agg(ha, hb, src, dst, zrows)
    ha, hb = _layer(sa, sb, cnt0, cnt1, ha, hb, W2l, b2l.reshape(1, -1), W2r)

    sa, sb = _sc_agg(ha, hb, src, dst, zrows)
    pooled = _final(bounds, sa, sb, cnt0, cnt1, ha, hb,
                    W3l, b3l.reshape(1, -1), W3r,
                    dec_W1, dec_b1.reshape(1, -1),
                    dec_W2, dec_b2.reshape(1, -1))

    return pooled.reshape(-1, 12)


# speedup vs baseline: 1.3395x; 1.3395x over previous
import functools

import jax
import jax.numpy as jnp
from jax import lax
from jax.experimental import pallas as pl
from jax.experimental.pallas import tpu as pltpu
from jax.experimental.pallas import tpu_sc as plsc

N_NODES = 50000
N_EDGES = 800000
D_IN = 128
D_HID = 64
HALF = 32
N_GRAPHS = 64

N_TILES = 16
N_CORES = 2
ROWS_PER_TILE = N_NODES // N_TILES
E_PER_TILE = N_EDGES // N_TILES
E_CHUNK = 400
N_CHUNKS = E_PER_TILE // E_CHUNK
N_PAIRS = (N_CHUNKS + 1) // 2

E_PER_CTILE = N_EDGES // (N_CORES * N_TILES)
EC_CHUNK = 200
NC_CHUNKS = E_PER_CTILE // EC_CHUNK

NB = 2000
N_BLOCKS = N_NODES // NB
NBL = 5000
NL_BLOCKS = N_NODES // NBL

_MESH = plsc.VectorSubcoreMesh(core_axis_name="c", subcore_axis_name="s",
                               num_cores=N_CORES, num_subcores=N_TILES)


def _sc_counts_body(eidx, zrows, ones32, cnt0, cnt1, dst_v, ones_v, cnt_sh):
    cid = lax.axis_index("c")
    sid = lax.axis_index("s")
    base = sid * ROWS_PER_TILE

    pltpu.sync_copy(zrows.at[pl.ds(base, ROWS_PER_TILE)],
                    cnt_sh.at[pl.ds(base, ROWS_PER_TILE)])
    pltpu.sync_copy(ones32, ones_v)
    plsc.subcore_barrier()

    ebase0 = (cid * N_TILES + sid) * E_PER_CTILE

    def chunk(c, carry):
        pltpu.sync_copy(eidx.at[1, pl.ds(ebase0 + c * EC_CHUNK, EC_CHUNK)],
                        dst_v)
        pltpu.sync_copy(ones_v, cnt_sh.at[dst_v], add=True)
        return carry

    lax.fori_loop(0, NC_CHUNKS, chunk, 0)
    plsc.subcore_barrier()

    @pl.when(cid == 0)
    def _():
        pltpu.sync_copy(cnt_sh.at[pl.ds(base, ROWS_PER_TILE)],
                        cnt0.at[pl.ds(base, ROWS_PER_TILE)])

    @pl.when(cid == 1)
    def _():
        pltpu.sync_copy(cnt_sh.at[pl.ds(base, ROWS_PER_TILE)],
                        cnt1.at[pl.ds(base, ROWS_PER_TILE)])


_sc_counts = functools.partial(
    pl.kernel,
    out_type=(
        jax.ShapeDtypeStruct((N_NODES, HALF), jnp.float32),
        jax.ShapeDtypeStruct((N_NODES, HALF), jnp.float32),
    ),
    mesh=_MESH,
    scratch_types=[
        pltpu.VMEM((EC_CHUNK,), jnp.int32),
        pltpu.VMEM((EC_CHUNK, HALF), jnp.float32),
        pltpu.VMEM_SHARED((N_NODES, HALF), jnp.float32),
    ],
    compiler_params=pltpu.CompilerParams(use_tc_tiling_on_sc=False),
)(_sc_counts_body)


def _sc_agg_body(ha, hb, eidx, zrows,
                 suma, sumb,
                 eb_v0, eb_v1, rows_v0, rows_v1,
                 acc_sh, sem0, sem1):
    cid = lax.axis_index("c")
    sid = lax.axis_index("s")
    base = sid * ROWS_PER_TILE

    pltpu.sync_copy(zrows.at[pl.ds(base, ROWS_PER_TILE)],
                    acc_sh.at[pl.ds(base, ROWS_PER_TILE)])
    plsc.subcore_barrier()

    ebase0 = sid * E_PER_TILE

    def load_idx(c, eb_v):
        eb = ebase0 + c * E_CHUNK
        pltpu.sync_copy(eidx.at[:, pl.ds(eb, E_CHUNK)], eb_v)

    def start_gather(eb_v, rows_v, sem):
        @pl.when(cid == 0)
        def _():
            pltpu.async_copy(ha.at[eb_v.at[0]], rows_v, sem)

        @pl.when(cid == 1)
        def _():
            pltpu.async_copy(hb.at[eb_v.at[0]], rows_v, sem)

    def wait_gather(eb_v, rows_v, sem):
        @pl.when(cid == 0)
        def _():
            pltpu.make_async_copy(ha.at[eb_v.at[0]], rows_v, sem).wait()

        @pl.when(cid == 1)
        def _():
            pltpu.make_async_copy(hb.at[eb_v.at[0]], rows_v, sem).wait()

    load_idx(0, eb_v0)
    start_gather(eb_v0, rows_v0, sem0)
    load_idx(1, eb_v1)
    start_gather(eb_v1, rows_v1, sem1)

    def step(c, eb_v, rows_v, sem):
        @pl.when(c < N_CHUNKS)
        def _():
            wait_gather(eb_v, rows_v, sem)
            pltpu.sync_copy(rows_v, acc_sh.at[eb_v.at[1]], add=True)

            @pl.when(c + 2 < N_CHUNKS)
            def _():
                load_idx(c + 2, eb_v)
                start_gather(eb_v, rows_v, sem)

    def pair(i, carry):
        step(2 * i, eb_v0, rows_v0, sem0)
        step(2 * i + 1, eb_v1, rows_v1, sem1)
        return carry

    lax.fori_loop(0, N_PAIRS, pair, 0)
    plsc.subcore_barrier()

    @pl.when(cid == 0)
    def _():
        pltpu.sync_copy(acc_sh.at[pl.ds(base, ROWS_PER_TILE)],
                        suma.at[pl.ds(base, ROWS_PER_TILE)])

    @pl.when(cid == 1)
    def _():
        pltpu.sync_copy(acc_sh.at[pl.ds(base, ROWS_PER_TILE)],
                        sumb.at[pl.ds(base, ROWS_PER_TILE)])


_sc_agg = functools.partial(
    pl.kernel,
    out_type=(
        jax.ShapeDtypeStruct((N_NODES, HALF), jnp.float32),
        jax.ShapeDtypeStruct((N_NODES, HALF), jnp.float32),
    ),
    mesh=_MESH,
    scratch_types=[
        pltpu.VMEM((2, E_CHUNK), jnp.int32),
        pltpu.VMEM((2, E_CHUNK), jnp.int32),
        pltpu.VMEM((E_CHUNK, HALF), jnp.float32),
        pltpu.VMEM((E_CHUNK, HALF), jnp.float32),
        pltpu.VMEM_SHARED((N_NODES, HALF), jnp.float32),
        pltpu.SemaphoreType.DMA,
        pltpu.SemaphoreType.DMA,
    ],
    compiler_params=pltpu.CompilerParams(use_tc_tiling_on_sc=False),
)(_sc_agg_body)


def _enc_body(x_ref, w1, b1, w2, b2, batch_r, oa, ob, bnd):
    i = pl.program_id(0)
    h = jnp.dot(x_ref[...], w1[...], preferred_element_type=jnp.float32)
    h = jnp.maximum(h + b1[...], 0.0)
    h = jnp.dot(h, w2[...], preferred_element_type=jnp.float32)
    h = jnp.maximum(h + b2[...], 0.0)
    oa[...] = h[:, :HALF]
    ob[...] = h[:, HALF:]

    blk = batch_r[...][0]
    giota = lax.broadcasted_iota(jnp.int32, (N_GRAPHS + 1, 1), 0)
    part = jnp.sum((blk < giota).astype(jnp.float32), axis=1, keepdims=True)

    @pl.when(i == 0)
    def _():
        bnd[...] = jnp.zeros((N_GRAPHS + 1, 1), jnp.float32)

    bnd[...] += part


def _encoder(x, w1, b1, w2, b2, batch2d):
    return pl.pallas_call(
        _enc_body,
        grid=(N_BLOCKS,),
        in_specs=[
            pl.BlockSpec((NB, D_IN), lambda i: (i, 0)),
            pl.BlockSpec((D_IN, HALF), lambda i: (0, 0)),
            pl.BlockSpec((1, HALF), lambda i: (0, 0)),
            pl.BlockSpec((HALF, D_HID), lambda i: (0, 0)),
            pl.BlockSpec((1, D_HID), lambda i: (0, 0)),
            pl.BlockSpec((1, 1, NB), lambda i: (i, 0, 0)),
        ],
        out_specs=[
            pl.BlockSpec((NB, HALF), lambda i: (i, 0)),
            pl.BlockSpec((NB, HALF), lambda i: (i, 0)),
            pl.BlockSpec((N_GRAPHS + 1, 1), lambda i: (0, 0)),
        ],
        out_shape=[
            jax.ShapeDtypeStruct((N_NODES, HALF), jnp.float32),
            jax.ShapeDtypeStruct((N_NODES, HALF), jnp.float32),
            jax.ShapeDtypeStruct((N_GRAPHS + 1, 1), jnp.float32),
        ],
    )(x, w1, b1, w2, b2, batch2d)


def _sage_mix(sa, sb, c0, c1, ha, hb, wl, bl, wr):
    r = 1.0 / jnp.maximum(c0 + c1, 1.0)
    h = (jnp.dot(sa * r, wl[:HALF], preferred_element_type=jnp.float32)
         + jnp.dot(sb * r, wl[HALF:], preferred_element_type=jnp.float32)
         + bl
         + jnp.dot(ha, wr[:HALF], preferred_element_type=jnp.float32)
         + jnp.dot(hb, wr[HALF:], preferred_element_type=jnp.float32))
    return h


def _layer_body(sa, sb, c0, c1, ha, hb, wl, bl, wr, oa, ob):
    h = jnp.maximum(
        _sage_mix(sa[...], sb[...], c0[...], c1[...], ha[...], hb[...],
                  wl[...], bl[...], wr[...]), 0.0)
    oa[...] = h[:, :HALF]
    ob[...] = h[:, HALF:]


def _layer(sa, sb, c0, c1, ha, hb, wl, bl2, wr):
    return pl.pallas_call(
        _layer_body,
        grid=(NL_BLOCKS,),
        in_specs=[
            pl.BlockSpec((NBL, HALF), lambda i: (i, 0)),
            pl.BlockSpec((NBL, HALF), lambda i: (i, 0)),
            pl.BlockSpec((NBL, HALF), lambda i: (i, 0)),
            pl.BlockSpec((NBL, HALF), lambda i: (i, 0)),
            pl.BlockSpec((NBL, HALF), lambda i: (i, 0)),
            pl.BlockSpec((NBL, HALF), lambda i: (i, 0)),
            pl.BlockSpec((D_HID, D_HID), lambda i: (0, 0)),
            pl.BlockSpec((1, D_HID), lambda i: (0, 0)),
            pl.BlockSpec((D_HID, D_HID), lambda i: (0, 0)),
        ],
        out_specs=[
            pl.BlockSpec((NBL, HALF), lambda i: (i, 0)),
            pl.BlockSpec((NBL, HALF), lambda i: (i, 0)),
        ],
        out_shape=[
            jax.ShapeDtypeStruct((N_NODES, HALF), jnp.float32),
            jax.ShapeDtypeStruct((N_NODES, HALF), jnp.float32),
        ],
    )(sa, sb, c0, c1, ha, hb, wl, bl2, wr)


def _final_body(bounds, sa, sb, c0, c1, ha, hb, w3l, b3l, w3r,
                dw1, db1, dw2, db2, out_ref):
    i = pl.program_id(0)
    row0 = i * NB
    h = jnp.maximum(
        _sage_mix(sa[...], sb[...], c0[...], c1[...], ha[...], hb[...],
                  w3l[...], b3l[...], w3r[...]), 0.0)
    d = jnp.maximum(
        jnp.dot(h, dw1[...], preferred_element_type=jnp.float32) + db1[...],
        0.0)
    o = jnp.dot(d, dw2[...], preferred_element_type=jnp.float32) + db2[...]

    @pl.when(i == 0)
    def _():
        out_ref[...] = jnp.full((N_GRAPHS, 48), -jnp.inf, jnp.float32)

    riota = lax.broadcasted_iota(jnp.int32, (NB, 1), 0) + row0
    for g in range(N_GRAPHS):
        s = bounds[g]
        e = bounds[g + 1]

        @pl.when((s < row0 + NB) & (e > row0))
        def _():
            m = jnp.max(jnp.where((riota >= s) & (riota < e), o, -jnp.inf),
                        axis=0, keepdims=True)
            out_ref[pl.ds(g, 1), :] = jnp.maximum(out_ref[pl.ds(g, 1), :], m)


def _final(bounds, sa, sb, c0, c1, ha, hb, w3l, b3l2, w3r,
           dw1, db12, dw2, db22):
    return pl.pallas_call(
        _final_body,
        grid_spec=pltpu.PrefetchScalarGridSpec(
            num_scalar_prefetch=1,
            grid=(N_BLOCKS,),
            in_specs=[
                pl.BlockSpec((NB, HALF), lambda i, b: (i, 0)),
                pl.BlockSpec((NB, HALF), lambda i, b: (i, 0)),
                pl.BlockSpec((NB, HALF), lambda i, b: (i, 0)),
                pl.BlockSpec((NB, HALF), lambda i, b: (i, 0)),
                pl.BlockSpec((NB, HALF), lambda i, b: (i, 0)),
                pl.BlockSpec((NB, HALF), lambda i, b: (i, 0)),
                pl.BlockSpec((D_HID, D_HID), lambda i, b: (0, 0)),
                pl.BlockSpec((1, D_HID), lambda i, b: (0, 0)),
                pl.BlockSpec((D_HID, D_HID), lambda i, b: (0, 0)),
                pl.BlockSpec((D_HID, HALF), lambda i, b: (0, 0)),
                pl.BlockSpec((1, HALF), lambda i, b: (0, 0)),
                pl.BlockSpec((HALF, 48), lambda i, b: (0, 0)),
                pl.BlockSpec((1, 48), lambda i, b: (0, 0)),
            ],
            out_specs=pl.BlockSpec((N_GRAPHS, 48), lambda i, b: (0, 0)),
        ),
        out_shape=jax.ShapeDtypeStruct((N_GRAPHS, 48), jnp.float32),
    )(bounds, sa, sb, c0, c1, ha, hb, w3l, b3l2, w3r, dw1, db12, dw2, db22)


def kernel(x, edge_index, batch, enc_W1, enc_b1, enc_W2, enc_b2,
           W1l, b1l, W1r, W2l, b2l, W2r, W3l, b3l, W3r,
           dec_W1, dec_b1, dec_W2, dec_b2):
    zrows = jnp.zeros((N_NODES, HALF), jnp.float32)
    ones32 = jnp.ones((EC_CHUNK, HALF), jnp.float32)
    batch2d = batch.reshape(N_BLOCKS, 1, NB)

    cnt0, cnt1 = _sc_counts(edge_index, zrows, ones32)

    ha, hb, bounds_f = _encoder(x, enc_W1, enc_b1.reshape(1, -1),
                                enc_W2, enc_b2.reshape(1, -1), batch2d)
    bounds = bounds_f[:, 0].astype(jnp.int32)

    sa, sb = _sc_agg(ha, hb, edge_index, zrows)
    ha, hb = _layer(sa, sb, cnt0, cnt1, ha, hb, W1l, b1l.reshape(1, -1), W1r)

    sa, sb = _sc_agg(ha, hb, edge_index, zrows)
    ha, hb = _layer(sa, sb, cnt0, cnt1, ha, hb, W2l, b2l.reshape(1, -1), W2r)

    sa, sb = _sc_agg(ha, hb, edge_index, zrows)
    pooled = _final(bounds, sa, sb, cnt0, cnt1, ha, hb,
                    W3l, b3l.reshape(1, -1), W3r,
                    dec_W1, dec_b1.reshape(1, -1),
                    dec_W2, dec_b2.reshape(1, -1))

    return pooled.reshape(-1, 12)


# speedup vs baseline: 1.4624x; 1.0918x over previous
import functools

import jax
import jax.numpy as jnp
from jax import lax
from jax.experimental import pallas as pl
from jax.experimental.pallas import tpu as pltpu
from jax.experimental.pallas import tpu_sc as plsc

N_NODES = 50000
N_EDGES = 800000
D_IN = 128
D_HID = 64
HALF = 32
N_GRAPHS = 64

N_TILES = 16
N_CORES = 2
ROWS_PER_TILE = N_NODES // N_TILES
E_PER_TILE = N_EDGES // N_TILES
E_CHUNK = 296
N_FULL = 168
E_TAIL = E_PER_TILE - N_FULL * E_CHUNK
N_TRIPLES = N_FULL // 3

CW = 8
E_PER_CTILE = N_EDGES // (N_CORES * N_TILES)
EC_CHUNK = 1000
NC_CHUNKS = E_PER_CTILE // EC_CHUNK
NC_PAIRS = (NC_CHUNKS + 1) // 2

NB = 2000
N_BLOCKS = N_NODES // NB
NBL = 5000
NL_BLOCKS = N_NODES // NBL

_MESH = plsc.VectorSubcoreMesh(core_axis_name="c", subcore_axis_name="s",
                               num_cores=N_CORES, num_subcores=N_TILES)


def _sc_counts_body(eidx, z8, ones8, cnt0, cnt1,
                    dst_v0, dst_v1, ones_v, cnt_sh, lsem0, lsem1):
    cid = lax.axis_index("c")
    sid = lax.axis_index("s")
    base = sid * ROWS_PER_TILE

    pltpu.sync_copy(z8.at[pl.ds(base, ROWS_PER_TILE)],
                    cnt_sh.at[pl.ds(base, ROWS_PER_TILE)])
    pltpu.sync_copy(ones8, ones_v)
    plsc.subcore_barrier()

    ebase0 = (cid * N_TILES + sid) * E_PER_CTILE

    def start_load(c, dst_v, lsem):
        pltpu.async_copy(
            eidx.at[1, pl.ds(ebase0 + c * EC_CHUNK, EC_CHUNK)], dst_v, lsem)

    def wait_load(c, dst_v, lsem):
        pltpu.make_async_copy(
            eidx.at[1, pl.ds(ebase0 + c * EC_CHUNK, EC_CHUNK)], dst_v,
            lsem).wait()

    start_load(0, dst_v0, lsem0)

    def step(c, dst_v, lsem, dst_o, lsem_o):
        @pl.when(c < NC_CHUNKS)
        def _():
            @pl.when(c + 1 < NC_CHUNKS)
            def _():
                start_load(c + 1, dst_o, lsem_o)

            wait_load(c, dst_v, lsem)
            pltpu.sync_copy(ones_v, cnt_sh.at[dst_v], add=True)

    def pair(i, carry):
        step(2 * i, dst_v0, lsem0, dst_v1, lsem1)
        step(2 * i + 1, dst_v1, lsem1, dst_v0, lsem0)
        return carry

    lax.fori_loop(0, NC_PAIRS, pair, 0)
    plsc.subcore_barrier()

    @pl.when(cid == 0)
    def _():
        pltpu.sync_copy(cnt_sh.at[pl.ds(base, ROWS_PER_TILE)],
                        cnt0.at[pl.ds(base, ROWS_PER_TILE)])

    @pl.when(cid == 1)
    def _():
        pltpu.sync_copy(cnt_sh.at[pl.ds(base, ROWS_PER_TILE)],
                        cnt1.at[pl.ds(base, ROWS_PER_TILE)])


_sc_counts = functools.partial(
    pl.kernel,
    out_type=(
        jax.ShapeDtypeStruct((N_NODES, CW), jnp.float32),
        jax.ShapeDtypeStruct((N_NODES, CW), jnp.float32),
    ),
    mesh=_MESH,
    scratch_types=[
        pltpu.VMEM((EC_CHUNK,), jnp.int32),
        pltpu.VMEM((EC_CHUNK,), jnp.int32),
        pltpu.VMEM((EC_CHUNK, CW), jnp.float32),
        pltpu.VMEM_SHARED((N_NODES, CW), jnp.float32),
        pltpu.SemaphoreType.DMA,
        pltpu.SemaphoreType.DMA,
    ],
    compiler_params=pltpu.CompilerParams(use_tc_tiling_on_sc=False),
)(_sc_counts_body)


def _sc_agg_body(ha, hb, eidx, zrows,
                 suma, sumb,
                 eb_v0, eb_v1, eb_v2, eb_t, rows_v0, rows_v1, rows_v2,
                 acc_sh, gsem0, gsem1, gsem2, ssem0, ssem1, ssem2, gsemt):
    cid = lax.axis_index("c")
    sid = lax.axis_index("s")
    base = sid * ROWS_PER_TILE

    pltpu.sync_copy(zrows.at[pl.ds(base, ROWS_PER_TILE)],
                    acc_sh.at[pl.ds(base, ROWS_PER_TILE)])
    plsc.subcore_barrier()

    ebase0 = sid * E_PER_TILE
    sets = ((eb_v0, rows_v0, gsem0, ssem0),
            (eb_v1, rows_v1, gsem1, ssem1),
            (eb_v2, rows_v2, gsem2, ssem2))

    def load_idx(c, eb_v):
        pltpu.sync_copy(eidx.at[:, pl.ds(ebase0 + c * E_CHUNK, E_CHUNK)],
                        eb_v)

    def start_gather(eb_v, rows_v, gsem):
        @pl.when(cid == 0)
        def _():
            pltpu.async_copy(ha.at[eb_v.at[0]], rows_v, gsem)

        @pl.when(cid == 1)
        def _():
            pltpu.async_copy(hb.at[eb_v.at[0]], rows_v, gsem)

    def wait_gather(eb_v, rows_v, gsem):
        @pl.when(cid == 0)
        def _():
            pltpu.make_async_copy(ha.at[eb_v.at[0]], rows_v, gsem).wait()

        @pl.when(cid == 1)
        def _():
            pltpu.make_async_copy(hb.at[eb_v.at[0]], rows_v, gsem).wait()

    def start_scatter(eb_v, rows_v, ssem):
        pltpu.async_copy(rows_v, acc_sh.at[eb_v.at[1]], ssem, add=True)

    def wait_scatter(eb_v, rows_v, ssem):
        pltpu.make_async_copy(rows_v, acc_sh.at[eb_v.at[1]], ssem).wait()

    load_idx(0, eb_v0)
    start_gather(eb_v0, rows_v0, gsem0)
    load_idx(1, eb_v1)
    start_gather(eb_v1, rows_v1, gsem1)

    tail_rows = rows_v0.at[pl.ds(0, E_TAIL)]

    def step(c, k):
        eb_v, rows_v, gsem, ssem = sets[k]
        eb_p, rows_p, gsem_p, ssem_p = sets[(k + 2) % 3]

        wait_gather(eb_v, rows_v, gsem)
        start_scatter(eb_v, rows_v, ssem)

        @pl.when(c <= N_FULL - 3)
        def _():
            @pl.when(c >= 1)
            def _():
                wait_scatter(eb_p, rows_p, ssem_p)
            load_idx(c + 2, eb_p)
            start_gather(eb_p, rows_p, gsem_p)

        @pl.when(c == N_FULL - 2)
        def _():
            wait_scatter(eb_p, rows_p, ssem_p)
            pltpu.sync_copy(
                eidx.at[:, pl.ds(ebase0 + N_FULL * E_CHUNK, E_TAIL)], eb_t)

            @pl.when(cid == 0)
            def _():
                pltpu.async_copy(ha.at[eb_t.at[0]], tail_rows, gsemt)

            @pl.when(cid == 1)
            def _():
                pltpu.async_copy(hb.at[eb_t.at[0]], tail_rows, gsemt)

    def triple(i, carry):
        step(3 * i, 0)
        step(3 * i + 1, 1)
        step(3 * i + 2, 2)
        return carry

    lax.fori_loop(0, N_TRIPLES, triple, 0)

    wait_scatter(eb_v1, rows_v1, ssem1)
    wait_scatter(eb_v2, rows_v2, ssem2)

    @pl.when(cid == 0)
    def _():
        pltpu.make_async_copy(ha.at[eb_t.at[0]], tail_rows, gsemt).wait()

    @pl.when(cid == 1)
    def _():
        pltpu.make_async_copy(hb.at[eb_t.at[0]], tail_rows, gsemt).wait()

    pltpu.sync_copy(tail_rows, acc_sh.at[eb_t.at[1]], add=True)
    plsc.subcore_barrier()

    @pl.when(cid == 0)
    def _():
        pltpu.sync_copy(acc_sh.at[pl.ds(base, ROWS_PER_TILE)],
                        suma.at[pl.ds(base, ROWS_PER_TILE)])

    @pl.when(cid == 1)
    def _():
        pltpu.sync_copy(acc_sh.at[pl.ds(base, ROWS_PER_TILE)],
                        sumb.at[pl.ds(base, ROWS_PER_TILE)])


_sc_agg = functools.partial(
    pl.kernel,
    out_type=(
        jax.ShapeDtypeStruct((N_NODES, HALF), jnp.float32),
        jax.ShapeDtypeStruct((N_NODES, HALF), jnp.float32),
    ),
    mesh=_MESH,
    scratch_types=[
        pltpu.VMEM((2, E_CHUNK), jnp.int32),
        pltpu.VMEM((2, E_CHUNK), jnp.int32),
        pltpu.VMEM((2, E_CHUNK), jnp.int32),
        pltpu.VMEM((2, E_TAIL), jnp.int32),
        pltpu.VMEM((E_CHUNK, HALF), jnp.float32),
        pltpu.VMEM((E_CHUNK, HALF), jnp.float32),
        pltpu.VMEM((E_CHUNK, HALF), jnp.float32),
        pltpu.VMEM_SHARED((N_NODES, HALF), jnp.float32),
        pltpu.SemaphoreType.DMA,
        pltpu.SemaphoreType.DMA,
        pltpu.SemaphoreType.DMA,
        pltpu.SemaphoreType.DMA,
        pltpu.SemaphoreType.DMA,
        pltpu.SemaphoreType.DMA,
        pltpu.SemaphoreType.DMA,
    ],
    compiler_params=pltpu.CompilerParams(use_tc_tiling_on_sc=False),
)(_sc_agg_body)


def _enc_body(x_ref, w1, b1, w2, b2, batch_r, oa, ob, bnd):
    i = pl.program_id(0)
    h = jnp.dot(x_ref[...], w1[...], preferred_element_type=jnp.float32)
    h = jnp.maximum(h + b1[...], 0.0)
    h = jnp.dot(h, w2[...], preferred_element_type=jnp.float32)
    h = jnp.maximum(h + b2[...], 0.0)
    oa[...] = h[:, :HALF]
    ob[...] = h[:, HALF:]

    blk = batch_r[...][0]
    giota = lax.broadcasted_iota(jnp.int32, (N_GRAPHS + 1, 1), 0)
    part = jnp.sum((blk < giota).astype(jnp.float32), axis=1, keepdims=True)

    @pl.when(i == 0)
    def _():
        bnd[...] = jnp.zeros((N_GRAPHS + 1, 1), jnp.float32)

    bnd[...] += part


def _encoder(x, w1, b1, w2, b2, batch2d):
    return pl.pallas_call(
        _enc_body,
        grid=(N_BLOCKS,),
        in_specs=[
            pl.BlockSpec((NB, D_IN), lambda i: (i, 0)),
            pl.BlockSpec((D_IN, HALF), lambda i: (0, 0)),
            pl.BlockSpec((1, HALF), lambda i: (0, 0)),
            pl.BlockSpec((HALF, D_HID), lambda i: (0, 0)),
            pl.BlockSpec((1, D_HID), lambda i: (0, 0)),
            pl.BlockSpec((1, 1, NB), lambda i: (i, 0, 0)),
        ],
        out_specs=[
            pl.BlockSpec((NB, HALF), lambda i: (i, 0)),
            pl.BlockSpec((NB, HALF), lambda i: (i, 0)),
            pl.BlockSpec((N_GRAPHS + 1, 1), lambda i: (0, 0)),
        ],
        out_shape=[
            jax.ShapeDtypeStruct((N_NODES, HALF), jnp.float32),
            jax.ShapeDtypeStruct((N_NODES, HALF), jnp.float32),
            jax.ShapeDtypeStruct((N_GRAPHS + 1, 1), jnp.float32),
        ],
    )(x, w1, b1, w2, b2, batch2d)


def _sage_mix(sa, sb, c0, c1, ha, hb, wl, bl, wr):
    r = 1.0 / jnp.maximum(c0[:, :1] + c1[:, :1], 1.0)
    h = (jnp.dot(sa * r, wl[:HALF], preferred_element_type=jnp.float32)
         + jnp.dot(sb * r, wl[HALF:], preferred_element_type=jnp.float32)
         + bl
         + jnp.dot(ha, wr[:HALF], preferred_element_type=jnp.float32)
         + jnp.dot(hb, wr[HALF:], preferred_element_type=jnp.float32))
    return h


def _layer_body(sa, sb, c0, c1, ha, hb, wl, bl, wr, oa, ob):
    h = jnp.maximum(
        _sage_mix(sa[...], sb[...], c0[...], c1[...], ha[...], hb[...],
                  wl[...], bl[...], wr[...]), 0.0)
    oa[...] = h[:, :HALF]
    ob[...] = h[:, HALF:]


def _layer(sa, sb, c0, c1, ha, hb, wl, bl2, wr):
    return pl.pallas_call(
        _layer_body,
        grid=(NL_BLOCKS,),
        in_specs=[
            pl.BlockSpec((NBL, HALF), lambda i: (i, 0)),
            pl.BlockSpec((NBL, HALF), lambda i: (i, 0)),
            pl.BlockSpec((NBL, CW), lambda i: (i, 0)),
            pl.BlockSpec((NBL, CW), lambda i: (i, 0)),
            pl.BlockSpec((NBL, HALF), lambda i: (i, 0)),
            pl.BlockSpec((NBL, HALF), lambda i: (i, 0)),
            pl.BlockSpec((D_HID, D_HID), lambda i: (0, 0)),
            pl.BlockSpec((1, D_HID), lambda i: (0, 0)),
            pl.BlockSpec((D_HID, D_HID), lambda i: (0, 0)),
        ],
        out_specs=[
            pl.BlockSpec((NBL, HALF), lambda i: (i, 0)),
            pl.BlockSpec((NBL, HALF), lambda i: (i, 0)),
        ],
        out_shape=[
            jax.ShapeDtypeStruct((N_NODES, HALF), jnp.float32),
            jax.ShapeDtypeStruct((N_NODES, HALF), jnp.float32),
        ],
    )(sa, sb, c0, c1, ha, hb, wl, bl2, wr)


def _final_body(bounds, sa, sb, c0, c1, ha, hb, w3l, b3l, w3r,
                dw1, db1, dw2, db2, out_ref):
    i = pl.program_id(0)
    row0 = i * NB
    h = jnp.maximum(
        _sage_mix(sa[...], sb[...], c0[...], c1[...], ha[...], hb[...],
                  w3l[...], b3l[...], w3r[...]), 0.0)
    d = jnp.maximum(
        jnp.dot(h, dw1[...], preferred_element_type=jnp.float32) + db1[...],
        0.0)
    o = jnp.dot(d, dw2[...], preferred_element_type=jnp.float32) + db2[...]

    @pl.when(i == 0)
    def _():
        out_ref[...] = jnp.full((N_GRAPHS, 48), -jnp.inf, jnp.float32)

    riota = lax.broadcasted_iota(jnp.int32, (NB, 1), 0) + row0
    for g in range(N_GRAPHS):
        s = bounds[g]
        e = bounds[g + 1]

        @pl.when((s < row0 + NB) & (e > row0))
        def _():
            m = jnp.max(jnp.where((riota >= s) & (riota < e), o, -jnp.inf),
                        axis=0, keepdims=True)
            out_ref[pl.ds(g, 1), :] = jnp.maximum(out_ref[pl.ds(g, 1), :], m)


def _final(bounds, sa, sb, c0, c1, ha, hb, w3l, b3l2, w3r,
           dw1, db12, dw2, db22):
    return pl.pallas_call(
        _final_body,
        grid_spec=pltpu.PrefetchScalarGridSpec(
            num_scalar_prefetch=1,
            grid=(N_BLOCKS,),
            in_specs=[
                pl.BlockSpec((NB, HALF), lambda i, b: (i, 0)),
                pl.BlockSpec((NB, HALF), lambda i, b: (i, 0)),
                pl.BlockSpec((NB, CW), lambda i, b: (i, 0)),
                pl.BlockSpec((NB, CW), lambda i, b: (i, 0)),
                pl.BlockSpec((NB, HALF), lambda i, b: (i, 0)),
                pl.BlockSpec((NB, HALF), lambda i, b: (i, 0)),
                pl.BlockSpec((D_HID, D_HID), lambda i, b: (0, 0)),
                pl.BlockSpec((1, D_HID), lambda i, b: (0, 0)),
                pl.BlockSpec((D_HID, D_HID), lambda i, b: (0, 0)),
                pl.BlockSpec((D_HID, HALF), lambda i, b: (0, 0)),
                pl.BlockSpec((1, HALF), lambda i, b: (0, 0)),
                pl.BlockSpec((HALF, 48), lambda i, b: (0, 0)),
                pl.BlockSpec((1, 48), lambda i, b: (0, 0)),
            ],
            out_specs=pl.BlockSpec((N_GRAPHS, 48), lambda i, b: (0, 0)),
        ),
        out_shape=jax.ShapeDtypeStruct((N_GRAPHS, 48), jnp.float32),
    )(bounds, sa, sb, c0, c1, ha, hb, w3l, b3l2, w3r, dw1, db12, dw2, db22)


def kernel(x, edge_index, batch, enc_W1, enc_b1, enc_W2, enc_b2,
           W1l, b1l, W1r, W2l, b2l, W2r, W3l, b3l, W3r,
           dec_W1, dec_b1, dec_W2, dec_b2):
    zrows = jnp.zeros((N_NODES, HALF), jnp.float32)
    z8 = jnp.zeros((N_NODES, CW), jnp.float32)
    ones8 = jnp.ones((EC_CHUNK, CW), jnp.float32)
    batch2d = batch.reshape(N_BLOCKS, 1, NB)

    cnt0, cnt1 = _sc_counts(edge_index, z8, ones8)

    ha, hb, bounds_f = _encoder(x, enc_W1, enc_b1.reshape(1, -1),
                                enc_W2, enc_b2.reshape(1, -1), batch2d)
    bounds = bounds_f[:, 0].astype(jnp.int32)

    sa, sb = _sc_agg(ha, hb, edge_index, zrows)
    ha, hb = _layer(sa, sb, cnt0, cnt1, ha, hb, W1l, b1l.reshape(1, -1), W1r)

    sa, sb = _sc_agg(ha, hb, edge_index, zrows)
    ha, hb = _layer(sa, sb, cnt0, cnt1, ha, hb, W2l, b2l.reshape(1, -1), W2r)

    sa, sb = _sc_agg(ha, hb, edge_index, zrows)
    pooled = _final(bounds, sa, sb, cnt0, cnt1, ha, hb,
                    W3l, b3l.reshape(1, -1), W3r,
                    dec_W1, dec_b1.reshape(1, -1),
                    dec_W2, dec_b2.reshape(1, -1))

    return pooled.reshape(-1, 12)


# speedup vs baseline: 1.5004x; 1.0260x over previous
import functools

import jax
import jax.numpy as jnp
from jax import lax
from jax.experimental import pallas as pl
from jax.experimental.pallas import tpu as pltpu
from jax.experimental.pallas import tpu_sc as plsc

N_NODES = 50000
N_EDGES = 800000
D_IN = 128
D_HID = 64
HALF = 32
N_GRAPHS = 64

N_TILES = 16
N_CORES = 2
ROWS_PER_TILE = N_NODES // N_TILES
E_PER_TILE = N_EDGES // N_TILES
E_CHUNK = 296
N_FULL = 168
E_TAIL = E_PER_TILE - N_FULL * E_CHUNK
N_TRIPLES = N_FULL // 3

CW = 8
E_PER_CTILE = N_EDGES // N_TILES
EC_CHUNK = 1000
NC_CHUNKS = E_PER_CTILE // EC_CHUNK
NC_PAIRS = (NC_CHUNKS + 1) // 2

NB = 2000
N_BLOCKS = N_NODES // NB
NBL = 5000
NL_BLOCKS = N_NODES // NBL

_MESH = plsc.VectorSubcoreMesh(core_axis_name="c", subcore_axis_name="s",
                               num_cores=N_CORES, num_subcores=N_TILES)


def _sc_counts_body(eidx, z8, ones8, cnt0,
                    dst_v0, dst_v1, ones_v, cnt_sh, lsem0, lsem1):
    cid = lax.axis_index("c")
    sid = lax.axis_index("s")
    base = sid * ROWS_PER_TILE
    ebase0 = sid * E_PER_CTILE

    def start_load(c, dst_v, lsem):
        pltpu.async_copy(
            eidx.at[1, pl.ds(ebase0 + c * EC_CHUNK, EC_CHUNK)], dst_v, lsem)

    def wait_load(c, dst_v, lsem):
        pltpu.make_async_copy(
            eidx.at[1, pl.ds(ebase0 + c * EC_CHUNK, EC_CHUNK)], dst_v,
            lsem).wait()

    def step(c, dst_v, lsem, dst_o, lsem_o):
        @pl.when(c < NC_CHUNKS)
        def _():
            @pl.when(c + 1 < NC_CHUNKS)
            def _():
                start_load(c + 1, dst_o, lsem_o)

            wait_load(c, dst_v, lsem)
            pltpu.sync_copy(ones_v, cnt_sh.at[dst_v], add=True)

    def pair(i, carry):
        step(2 * i, dst_v0, lsem0, dst_v1, lsem1)
        step(2 * i + 1, dst_v1, lsem1, dst_v0, lsem0)
        return carry

    @pl.when(cid == 0)
    def _():
        pltpu.sync_copy(z8.at[pl.ds(base, ROWS_PER_TILE)],
                        cnt_sh.at[pl.ds(base, ROWS_PER_TILE)])
        pltpu.sync_copy(ones8, ones_v)

    plsc.subcore_barrier()

    @pl.when(cid == 0)
    def _():
        start_load(0, dst_v0, lsem0)
        lax.fori_loop(0, NC_PAIRS, pair, 0)

    plsc.subcore_barrier()

    @pl.when(cid == 0)
    def _():
        pltpu.sync_copy(cnt_sh.at[pl.ds(base, ROWS_PER_TILE)],
                        cnt0.at[pl.ds(base, ROWS_PER_TILE)])


_sc_counts = functools.partial(
    pl.kernel,
    out_type=jax.ShapeDtypeStruct((N_NODES, CW), jnp.float32),
    mesh=_MESH,
    scratch_types=[
        pltpu.VMEM((EC_CHUNK,), jnp.int32),
        pltpu.VMEM((EC_CHUNK,), jnp.int32),
        pltpu.VMEM((EC_CHUNK, CW), jnp.float32),
        pltpu.VMEM_SHARED((N_NODES, CW), jnp.float32),
        pltpu.SemaphoreType.DMA,
        pltpu.SemaphoreType.DMA,
    ],
    compiler_params=pltpu.CompilerParams(use_tc_tiling_on_sc=False),
)(_sc_counts_body)


def _sc_agg_body(ha, hb, eidx, zrows,
                 suma, sumb,
                 eb_v0, eb_v1, eb_v2, eb_t, rows_v0, rows_v1, rows_v2,
                 acc_sh, gsem0, gsem1, gsem2, ssem0, ssem1, ssem2, gsemt):
    cid = lax.axis_index("c")
    sid = lax.axis_index("s")
    base = sid * ROWS_PER_TILE

    pltpu.sync_copy(zrows.at[pl.ds(base, ROWS_PER_TILE)],
                    acc_sh.at[pl.ds(base, ROWS_PER_TILE)])
    plsc.subcore_barrier()

    ebase0 = sid * E_PER_TILE
    sets = ((eb_v0, rows_v0, gsem0, ssem0),
            (eb_v1, rows_v1, gsem1, ssem1),
            (eb_v2, rows_v2, gsem2, ssem2))

    def load_idx(c, eb_v):
        pltpu.sync_copy(eidx.at[:, pl.ds(ebase0 + c * E_CHUNK, E_CHUNK)],
                        eb_v)

    def start_gather(eb_v, rows_v, gsem):
        @pl.when(cid == 0)
        def _():
            pltpu.async_copy(ha.at[eb_v.at[0]], rows_v, gsem)

        @pl.when(cid == 1)
        def _():
            pltpu.async_copy(hb.at[eb_v.at[0]], rows_v, gsem)

    def wait_gather(eb_v, rows_v, gsem):
        @pl.when(cid == 0)
        def _():
            pltpu.make_async_copy(ha.at[eb_v.at[0]], rows_v, gsem).wait()

        @pl.when(cid == 1)
        def _():
            pltpu.make_async_copy(hb.at[eb_v.at[0]], rows_v, gsem).wait()

    def start_scatter(eb_v, rows_v, ssem):
        pltpu.async_copy(rows_v, acc_sh.at[eb_v.at[1]], ssem, add=True)

    def wait_scatter(eb_v, rows_v, ssem):
        pltpu.make_async_copy(rows_v, acc_sh.at[eb_v.at[1]], ssem).wait()

    load_idx(0, eb_v0)
    start_gather(eb_v0, rows_v0, gsem0)
    load_idx(1, eb_v1)
    start_gather(eb_v1, rows_v1, gsem1)

    tail_rows = rows_v0.at[pl.ds(0, E_TAIL)]

    def step(c, k):
        eb_v, rows_v, gsem, ssem = sets[k]
        eb_p, rows_p, gsem_p, ssem_p = sets[(k + 2) % 3]

        wait_gather(eb_v, rows_v, gsem)
        start_scatter(eb_v, rows_v, ssem)

        @pl.when(c <= N_FULL - 3)
        def _():
            @pl.when(c >= 1)
            def _():
                wait_scatter(eb_p, rows_p, ssem_p)
            load_idx(c + 2, eb_p)
            start_gather(eb_p, rows_p, gsem_p)

        @pl.when(c == N_FULL - 2)
        def _():
            wait_scatter(eb_p, rows_p, ssem_p)
            pltpu.sync_copy(
                eidx.at[:, pl.ds(ebase0 + N_FULL * E_CHUNK, E_TAIL)], eb_t)

            @pl.when(cid == 0)
            def _():
                pltpu.async_copy(ha.at[eb_t.at[0]], tail_rows, gsemt)

            @pl.when(cid == 1)
            def _():
                pltpu.async_copy(hb.at[eb_t.at[0]], tail_rows, gsemt)

    def triple(i, carry):
        step(3 * i, 0)
        step(3 * i + 1, 1)
        step(3 * i + 2, 2)
        return carry

    lax.fori_loop(0, N_TRIPLES, triple, 0)

    wait_scatter(eb_v1, rows_v1, ssem1)
    wait_scatter(eb_v2, rows_v2, ssem2)

    @pl.when(cid == 0)
    def _():
        pltpu.make_async_copy(ha.at[eb_t.at[0]], tail_rows, gsemt).wait()

    @pl.when(cid == 1)
    def _():
        pltpu.make_async_copy(hb.at[eb_t.at[0]], tail_rows, gsemt).wait()

    pltpu.sync_copy(tail_rows, acc_sh.at[eb_t.at[1]], add=True)
    plsc.subcore_barrier()

    @pl.when(cid == 0)
    def _():
        pltpu.sync_copy(acc_sh.at[pl.ds(base, ROWS_PER_TILE)],
                        suma.at[pl.ds(base, ROWS_PER_TILE)])

    @pl.when(cid == 1)
    def _():
        pltpu.sync_copy(acc_sh.at[pl.ds(base, ROWS_PER_TILE)],
                        sumb.at[pl.ds(base, ROWS_PER_TILE)])


_sc_agg = functools.partial(
    pl.kernel,
    out_type=(
        jax.ShapeDtypeStruct((N_NODES, HALF), jnp.float32),
        jax.ShapeDtypeStruct((N_NODES, HALF), jnp.float32),
    ),
    mesh=_MESH,
    scratch_types=[
        pltpu.VMEM((2, E_CHUNK), jnp.int32),
        pltpu.VMEM((2, E_CHUNK), jnp.int32),
        pltpu.VMEM((2, E_CHUNK), jnp.int32),
        pltpu.VMEM((2, E_TAIL), jnp.int32),
        pltpu.VMEM((E_CHUNK, HALF), jnp.float32),
        pltpu.VMEM((E_CHUNK, HALF), jnp.float32),
        pltpu.VMEM((E_CHUNK, HALF), jnp.float32),
        pltpu.VMEM_SHARED((N_NODES, HALF), jnp.float32),
        pltpu.SemaphoreType.DMA,
        pltpu.SemaphoreType.DMA,
        pltpu.SemaphoreType.DMA,
        pltpu.SemaphoreType.DMA,
        pltpu.SemaphoreType.DMA,
        pltpu.SemaphoreType.DMA,
        pltpu.SemaphoreType.DMA,
    ],
    compiler_params=pltpu.CompilerParams(use_tc_tiling_on_sc=False),
)(_sc_agg_body)


def _enc_body(x_ref, w1, b1, w2, b2, batch_r, oa, ob, bnd):
    i = pl.program_id(0)
    h = jnp.dot(x_ref[...], w1[...], preferred_element_type=jnp.float32)
    h = jnp.maximum(h + b1[...], 0.0)
    h = jnp.dot(h, w2[...], preferred_element_type=jnp.float32)
    h = jnp.maximum(h + b2[...], 0.0)
    oa[...] = h[:, :HALF]
    ob[...] = h[:, HALF:]

    blk = batch_r[...][0]
    giota = lax.broadcasted_iota(jnp.int32, (N_GRAPHS + 1, 1), 0)
    part = jnp.sum((blk < giota).astype(jnp.float32), axis=1, keepdims=True)

    @pl.when(i == 0)
    def _():
        bnd[...] = jnp.zeros((N_GRAPHS + 1, 1), jnp.float32)

    bnd[...] += part


def _encoder(x, w1, b1, w2, b2, batch2d):
    return pl.pallas_call(
        _enc_body,
        grid=(N_BLOCKS,),
        in_specs=[
            pl.BlockSpec((NB, D_IN), lambda i: (i, 0)),
            pl.BlockSpec((D_IN, HALF), lambda i: (0, 0)),
            pl.BlockSpec((1, HALF), lambda i: (0, 0)),
            pl.BlockSpec((HALF, D_HID), lambda i: (0, 0)),
            pl.BlockSpec((1, D_HID), lambda i: (0, 0)),
            pl.BlockSpec((1, 1, NB), lambda i: (i, 0, 0)),
        ],
        out_specs=[
            pl.BlockSpec((NB, HALF), lambda i: (i, 0)),
            pl.BlockSpec((NB, HALF), lambda i: (i, 0)),
            pl.BlockSpec((N_GRAPHS + 1, 1), lambda i: (0, 0)),
        ],
        out_shape=[
            jax.ShapeDtypeStruct((N_NODES, HALF), jnp.float32),
            jax.ShapeDtypeStruct((N_NODES, HALF), jnp.float32),
            jax.ShapeDtypeStruct((N_GRAPHS + 1, 1), jnp.float32),
        ],
    )(x, w1, b1, w2, b2, batch2d)


def _sage_mix(sa, sb, c0, ha, hb, wl, bl, wr):
    r = 1.0 / jnp.maximum(c0[:, :1], 1.0)
    h = (jnp.dot(sa * r, wl[:HALF], preferred_element_type=jnp.float32)
         + jnp.dot(sb * r, wl[HALF:], preferred_element_type=jnp.float32)
         + bl
         + jnp.dot(ha, wr[:HALF], preferred_element_type=jnp.float32)
         + jnp.dot(hb, wr[HALF:], preferred_element_type=jnp.float32))
    return h


def _layer_body(sa, sb, c0, ha, hb, wl, bl, wr, oa, ob):
    h = jnp.maximum(
        _sage_mix(sa[...], sb[...], c0[...], ha[...], hb[...],
                  wl[...], bl[...], wr[...]), 0.0)
    oa[...] = h[:, :HALF]
    ob[...] = h[:, HALF:]


def _layer(sa, sb, c0, ha, hb, wl, bl2, wr):
    return pl.pallas_call(
        _layer_body,
        grid=(NL_BLOCKS,),
        in_specs=[
            pl.BlockSpec((NBL, HALF), lambda i: (i, 0)),
            pl.BlockSpec((NBL, HALF), lambda i: (i, 0)),
            pl.BlockSpec((NBL, CW), lambda i: (i, 0)),
            pl.BlockSpec((NBL, HALF), lambda i: (i, 0)),
            pl.BlockSpec((NBL, HALF), lambda i: (i, 0)),
            pl.BlockSpec((D_HID, D_HID), lambda i: (0, 0)),
            pl.BlockSpec((1, D_HID), lambda i: (0, 0)),
            pl.BlockSpec((D_HID, D_HID), lambda i: (0, 0)),
        ],
        out_specs=[
            pl.BlockSpec((NBL, HALF), lambda i: (i, 0)),
            pl.BlockSpec((NBL, HALF), lambda i: (i, 0)),
        ],
        out_shape=[
            jax.ShapeDtypeStruct((N_NODES, HALF), jnp.float32),
            jax.ShapeDtypeStruct((N_NODES, HALF), jnp.float32),
        ],
    )(sa, sb, c0, ha, hb, wl, bl2, wr)


def _final_body(bounds, sa, sb, c0, ha, hb, w3l, b3l, w3r,
                dw1, db1, dw2, db2, out_ref):
    i = pl.program_id(0)
    row0 = i * NB
    h = jnp.maximum(
        _sage_mix(sa[...], sb[...], c0[...], ha[...], hb[...],
                  w3l[...], b3l[...], w3r[...]), 0.0)
    d = jnp.maximum(
        jnp.dot(h, dw1[...], preferred_element_type=jnp.float32) + db1[...],
        0.0)
    o = jnp.dot(d, dw2[...], preferred_element_type=jnp.float32) + db2[...]

    @pl.when(i == 0)
    def _():
        out_ref[...] = jnp.full((N_GRAPHS, 48), -jnp.inf, jnp.float32)

    riota = lax.broadcasted_iota(jnp.int32, (NB, 1), 0) + row0
    for g in range(N_GRAPHS):
        s = bounds[g]
        e = bounds[g + 1]

        @pl.when((s < row0 + NB) & (e > row0))
        def _():
            m = jnp.max(jnp.where((riota >= s) & (riota < e), o, -jnp.inf),
                        axis=0, keepdims=True)
            out_ref[pl.ds(g, 1), :] = jnp.maximum(out_ref[pl.ds(g, 1), :], m)


def _final(bounds, sa, sb, c0, ha, hb, w3l, b3l2, w3r,
           dw1, db12, dw2, db22):
    return pl.pallas_call(
        _final_body,
        grid_spec=pltpu.PrefetchScalarGridSpec(
            num_scalar_prefetch=1,
            grid=(N_BLOCKS,),
            in_specs=[
                pl.BlockSpec((NB, HALF), lambda i, b: (i, 0)),
                pl.BlockSpec((NB, HALF), lambda i, b: (i, 0)),
                pl.BlockSpec((NB, CW), lambda i, b: (i, 0)),
                pl.BlockSpec((NB, HALF), lambda i, b: (i, 0)),
                pl.BlockSpec((NB, HALF), lambda i, b: (i, 0)),
                pl.BlockSpec((D_HID, D_HID), lambda i, b: (0, 0)),
                pl.BlockSpec((1, D_HID), lambda i, b: (0, 0)),
                pl.BlockSpec((D_HID, D_HID), lambda i, b: (0, 0)),
                pl.BlockSpec((D_HID, HALF), lambda i, b: (0, 0)),
                pl.BlockSpec((1, HALF), lambda i, b: (0, 0)),
                pl.BlockSpec((HALF, 48), lambda i, b: (0, 0)),
                pl.BlockSpec((1, 48), lambda i, b: (0, 0)),
            ],
            out_specs=pl.BlockSpec((N_GRAPHS, 48), lambda i, b: (0, 0)),
        ),
        out_shape=jax.ShapeDtypeStruct((N_GRAPHS, 48), jnp.float32),
    )(bounds, sa, sb, c0, ha, hb, w3l, b3l2, w3r, dw1, db12, dw2, db22)


def kernel(x, edge_index, batch, enc_W1, enc_b1, enc_W2, enc_b2,
           W1l, b1l, W1r, W2l, b2l, W2r, W3l, b3l, W3r,
           dec_W1, dec_b1, dec_W2, dec_b2):
    zrows = jnp.zeros((N_NODES, HALF), jnp.float32)
    z8 = jnp.zeros((N_NODES, CW), jnp.float32)
    ones8 = jnp.ones((EC_CHUNK, CW), jnp.float32)
    batch2d = batch.reshape(N_BLOCKS, 1, NB)

    cnt0 = _sc_counts(edge_index, z8, ones8)

    ha, hb, bounds_f = _encoder(x, enc_W1, enc_b1.reshape(1, -1),
                                enc_W2, enc_b2.reshape(1, -1), batch2d)
    bounds = bounds_f[:, 0].astype(jnp.int32)

    sa, sb = _sc_agg(ha, hb, edge_index, zrows)
    ha, hb = _layer(sa, sb, cnt0, ha, hb, W1l, b1l.reshape(1, -1), W1r)

    sa, sb = _sc_agg(ha, hb, edge_index, zrows)
    ha, hb = _layer(sa, sb, cnt0, ha, hb, W2l, b2l.reshape(1, -1), W2r)

    sa, sb = _sc_agg(ha, hb, edge_index, zrows)
    pooled = _final(bounds, sa, sb, cnt0, ha, hb,
                    W3l, b3l.reshape(1, -1), W3r,
                    dec_W1, dec_b1.reshape(1, -1),
                    dec_W2, dec_b2.reshape(1, -1))

    return pooled.reshape(-1, 12)
